# Initial kernel scaffold; baseline (speedup 1.0000x reference)
import jax
import jax.numpy as jnp
from jax.experimental import pallas as pl


def kernel(x, W):
    raise NotImplementedError("write your pallas kernel here")


# speedup vs baseline: 1.5722x; 1.5722x over previous
import functools
import math

import jax
import jax.numpy as jnp
from jax import lax
from jax.experimental import pallas as pl
from jax.experimental.pallas import tpu as pltpu
from jax.experimental.pallas import tpu_sc as plsc

_D = 2048
_N = 8192
_NC, _NS = 2, 16
_NW = _NC * _NS
_PER_W = _N // _NW
_C = 16
_NCHUNK = _PER_W // _C
_NBUF = 3
_LANES = 16
_SCALE = math.sqrt(float(_D))


def _emb_body(x_hbm, w_hbm, out_hbm, idx_v, rows, *sems):
  gsem = sems[:_NBUF]
  ssem = sems[_NBUF:]
  wid = lax.axis_index("s") * _NC + lax.axis_index("c")
  base = wid * _PER_W
  pltpu.sync_copy(x_hbm.at[pl.ds(base, _PER_W)], idx_v)

  gathers = [None] * _NCHUNK
  scatters = [None] * _NCHUNK

  def start_gather(c):
    b = c % _NBUF
    gathers[c] = pltpu.async_copy(
        w_hbm.at[idx_v.at[pl.ds(c * _C, _C)]], rows.at[b], gsem[b])

  start_gather(0)
  start_gather(1)
  for c in range(_NCHUNK):
    b = c % _NBUF
    gathers[c].wait()

    @plsc.parallel_loop(0, _C * _D, step=_LANES, unroll=8)
    def _scale(i, b=b):
      r = i // _D
      jj = i - r * _D
      rows[b, r, pl.ds(jj, _LANES)] = rows[b, r, pl.ds(jj, _LANES)] * _SCALE

    scatters[c] = pltpu.async_copy(
        rows.at[b], out_hbm.at[pl.ds(base + c * _C, _C)], ssem[b])
    nxt = c + 2
    if nxt < _NCHUNK:
      if nxt >= _NBUF:
        scatters[nxt - _NBUF].wait()
      start_gather(nxt)
  for c in range(_NCHUNK - _NBUF, _NCHUNK):
    scatters[c].wait()


@functools.lru_cache(maxsize=None)
def _build_emb():
  mesh = plsc.VectorSubcoreMesh(
      core_axis_name="c", subcore_axis_name="s",
      num_cores=_NC, num_subcores=_NS,
  )
  return pl.kernel(
      _emb_body,
      out_type=jax.ShapeDtypeStruct((_N, _D), jnp.float32),
      mesh=mesh,
      scratch_types=(
          [pltpu.VMEM((_PER_W,), jnp.int32),
           pltpu.VMEM((_NBUF, _C, _D), jnp.float32)]
          + [pltpu.SemaphoreType.DMA] * (2 * _NBUF)
      ),
  )


@jax.jit
def kernel(x, W):
  out = _build_emb()(x.reshape(_N), W)
  return out.reshape(*x.shape, _D)
